# baseline (speedup 1.0000x reference)
import jax
import jax.numpy as jnp
from jax import lax
from jax.experimental import pallas as pl
from jax.experimental.pallas import tpu as pltpu
from jax.experimental.pallas import tpu_sc as plsc

_BINS = 2048
_NC = 2
_NS = 16
_NW = _NC * _NS
_L = 16
_CHUNK = 32768
_U = 8
_STRIDE = _BINS + 2


def _mesh():
    return plsc.VectorSubcoreMesh(core_axis_name="c", subcore_axis_name="s",
                                  num_cores=_NC, num_subcores=_NS)


def _minmax_body(x_hbm, pmin_hbm, pmax_hbm,
                 buf0, buf1, stage, lsem0, lsem1):
    wid = lax.axis_index("c") * _NS + lax.axis_index("s")
    nrows = x_hbm.shape[0]
    rows_w = nrows // _NW
    rows_c = _CHUNK // 2048
    base = wid * rows_w
    n_chunks = rows_w // rows_c
    lsem = (lsem0, lsem1)
    bufs = (buf0, buf1)

    def load(ci, s):
        sl = pl.ds(base + ci * rows_c, rows_c)
        return pltpu.make_async_copy(x_hbm.at[sl, :], bufs[s], lsem[s])

    mns = [jnp.full((_L,), jnp.inf, jnp.float32) for _ in range(_U)]
    mxs = [jnp.full((_L,), -jnp.inf, jnp.float32) for _ in range(_U)]

    load(0, 0).start()
    for ci in range(n_chunks):
        s = ci & 1
        load(ci, s).wait()
        if ci + 1 < n_chunks:
            load(ci + 1, 1 - s).start()

        carry = (tuple(mns), tuple(mxs))

        @plsc.parallel_loop(0, _CHUNK // (_L * _U), 1, unroll=2, carry=carry)
        def _mm(i, c):
            ms, xs = c
            new_ms, new_xs = [], []
            for j in range(_U):
                k = i * _U + j
                v = bufs[s][k >> 7, pl.ds((k & 127) * _L, _L)]
                new_ms.append(jnp.minimum(ms[j], v))
                new_xs.append(jnp.maximum(xs[j], v))
            return tuple(new_ms), tuple(new_xs)

        mns, mxs = _mm

    minv, maxv = mns[0], mxs[0]
    for j in range(1, _U):
        minv = jnp.minimum(minv, mns[j])
        maxv = jnp.maximum(maxv, mxs[j])
    stage[...] = minv
    pltpu.sync_copy(stage, pmin_hbm.at[pl.ds(wid * _L, _L)])
    stage[...] = maxv
    pltpu.sync_copy(stage, pmax_hbm.at[pl.ds(wid * _L, _L)])


def _hist_body(x_hbm, pmin_hbm, pmax_hbm, xout_hbm, phist_hbm,
               buf0, buf1, mmbuf, hist, out_slot, lsem0, lsem1, ssem0, ssem1):
    wid = lax.axis_index("c") * _NS + lax.axis_index("s")
    nrows = x_hbm.shape[0]
    rows_w = nrows // _NW
    rows_c = _CHUNK // 2048
    base = wid * rows_w
    n_chunks = rows_w // rows_c
    lsem = (lsem0, lsem1)
    ssem = (ssem0, ssem1)
    bufs = (buf0, buf1)

    def load(ci, s):
        sl = pl.ds(base + ci * rows_c, rows_c)
        return pltpu.make_async_copy(x_hbm.at[sl, :], bufs[s], lsem[s])

    def store(ci, s):
        sl = pl.ds(base + ci * rows_c, rows_c)
        return pltpu.make_async_copy(bufs[s], xout_hbm.at[sl, :], ssem[s])

    load(0, 0).start()

    pltpu.sync_copy(pmin_hbm, mmbuf.at[0])
    pltpu.sync_copy(pmax_hbm, mmbuf.at[1])

    def mmstep(i, c):
        mn, mx = c
        return (jnp.minimum(mn, mmbuf[0, pl.ds(i * _L, _L)]),
                jnp.maximum(mx, mmbuf[1, pl.ds(i * _L, _L)]))

    minv, maxv = lax.fori_loop(
        0, _NW, mmstep,
        (jnp.full((_L,), jnp.inf, jnp.float32),
         jnp.full((_L,), -jnp.inf, jnp.float32)))
    xmin, xmax = minv[0], maxv[0]
    for j in range(1, _L):
        xmin = jnp.minimum(xmin, minv[j])
        xmax = jnp.maximum(xmax, maxv[j])
    rng = jnp.where(xmax > xmin, xmax - xmin, jnp.float32(1.0))
    rngv = jnp.full((_L,), 1.0, jnp.float32) * rng
    scale = jnp.full((_L,), float(_BINS), jnp.float32) / rngv
    noff = -(xmin * scale)

    def zstep(i, _):
        hist[pl.ds(i * _L, _L)] = jnp.zeros((_L,), jnp.float32)
        return 0

    lax.fori_loop(0, _L * _STRIDE // _L, zstep, 0)

    ones = jnp.ones((_L,), jnp.float32)
    lane_base = lax.broadcasted_iota(jnp.int32, (_L,), 0) * _STRIDE + 1

    for ci in range(n_chunks):
        s = ci & 1
        load(ci, s).wait()
        store(ci, s).start()
        if ci + 1 < n_chunks:
            if ci >= 1:
                store(ci - 1, 1 - s).wait()
            load(ci + 1, 1 - s).start()

        @plsc.parallel_loop(0, _CHUNK // _L, 1, unroll=_U)
        def _sc(i):
            v = bufs[s][i >> 7, pl.ds((i & 127) * _L, _L)]
            t = v * scale + noff
            idx = t.astype(jnp.int32) + lane_base
            plsc.addupdate_scatter(hist, [idx], ones)

    ov = plsc.load_gather(hist, [lane_base + _BINS])
    plsc.addupdate_scatter(hist, [jnp.full((_L,), _BINS - 1, jnp.int32)], ov)

    def rstep(k, _):
        acc = hist[pl.ds(k * _L, _L)]
        for l in range(1, _L):
            acc = acc + hist[pl.ds(l * _STRIDE + k * _L, _L)]
        hist[pl.ds(k * _L, _L)] = acc
        return 0

    store(n_chunks - 1, (n_chunks - 1) & 1).wait()
    store(n_chunks - 2, (n_chunks - 2) & 1).wait()
    lax.fori_loop(0, _BINS // _L, rstep, 0)
    pltpu.sync_copy(hist.at[pl.ds(0, _BINS)], phist_hbm.at[wid])


def _sc_minmax(x_flat):
    f = pl.kernel(
        _minmax_body,
        out_type=(jax.ShapeDtypeStruct((_NW * _L,), jnp.float32),
                  jax.ShapeDtypeStruct((_NW * _L,), jnp.float32)),
        mesh=_mesh(),
        scratch_types=[pltpu.VMEM((_CHUNK // 2048, 2048), jnp.float32),
                       pltpu.VMEM((_CHUNK // 2048, 2048), jnp.float32),
                       pltpu.VMEM((_L,), jnp.float32),
                       pltpu.SemaphoreType.DMA,
                       pltpu.SemaphoreType.DMA],
        compiler_params=pltpu.CompilerParams(needs_layout_passes=False, use_tc_tiling_on_sc=True),
    )
    return f(x_flat)


def _sc_hist(x_flat, pmin, pmax):
    f = pl.kernel(
        _hist_body,
        out_type=(jax.ShapeDtypeStruct(x_flat.shape, jnp.float32),
                  jax.ShapeDtypeStruct((_NW, _BINS), jnp.float32)),
        mesh=_mesh(),
        scratch_types=[pltpu.VMEM((_CHUNK // 2048, 2048), jnp.float32),
                       pltpu.VMEM((_CHUNK // 2048, 2048), jnp.float32),
                       pltpu.VMEM((2, _NW * _L), jnp.float32),
                       pltpu.VMEM((_L * _STRIDE,), jnp.float32),
                       pltpu.VMEM((_BINS,), jnp.float32),
                       pltpu.SemaphoreType.DMA,
                       pltpu.SemaphoreType.DMA,
                       pltpu.SemaphoreType.DMA,
                       pltpu.SemaphoreType.DMA],
        compiler_params=pltpu.CompilerParams(needs_layout_passes=False, use_tc_tiling_on_sc=True),
    )
    return f(x_flat, pmin, pmax)


def kernel(x):
    x_flat = x.reshape(-1, x.shape[-1])
    pmin, pmax = _sc_minmax(x_flat)
    x_out, phist = _sc_hist(x_flat, pmin, pmax)
    xmin = jnp.min(pmin)
    xmax = jnp.max(pmax)
    hist = jnp.sum(phist, axis=0)
    return (x_out.reshape(x.shape), xmin, xmax, hist)


# speedup vs baseline: 1.0063x; 1.0063x over previous
import jax
import jax.numpy as jnp
from jax import lax
from jax.experimental import pallas as pl
from jax.experimental.pallas import tpu as pltpu
from jax.experimental.pallas import tpu_sc as plsc

_BINS = 2048
_NC = 2
_NS = 16
_NW = _NC * _NS
_L = 16
_CHUNK = 32768
_U = 8
_STRIDE = _BINS + 2


def _mesh():
    return plsc.VectorSubcoreMesh(core_axis_name="c", subcore_axis_name="s",
                                  num_cores=_NC, num_subcores=_NS)


def _minmax_body(x_hbm, pmin_hbm, pmax_hbm,
                 buf0, buf1, stage, lsem0, lsem1):
    wid = lax.axis_index("c") * _NS + lax.axis_index("s")
    nrows = x_hbm.shape[0]
    rows_w = nrows // _NW
    rows_c = _CHUNK // 2048
    base = wid * rows_w
    n_chunks = rows_w // rows_c
    lsem = (lsem0, lsem1)
    bufs = (buf0, buf1)

    def load(ci, s):
        sl = pl.ds(base + ci * rows_c, rows_c)
        return pltpu.make_async_copy(x_hbm.at[sl, :], bufs[s], lsem[s])

    mns = [jnp.full((_L,), jnp.inf, jnp.float32) for _ in range(_U)]
    mxs = [jnp.full((_L,), -jnp.inf, jnp.float32) for _ in range(_U)]

    load(0, 0).start()
    for ci in range(n_chunks):
        s = ci & 1
        load(ci, s).wait()
        if ci + 1 < n_chunks:
            load(ci + 1, 1 - s).start()

        carry = (tuple(mns), tuple(mxs))

        @plsc.parallel_loop(0, _CHUNK // (_L * _U), 1, unroll=2, carry=carry)
        def _mm(i, c):
            ms, xs = c
            new_ms, new_xs = [], []
            for j in range(_U):
                k = i * _U + j
                v = bufs[s][k >> 7, pl.ds((k & 127) * _L, _L)]
                new_ms.append(jnp.minimum(ms[j], v))
                new_xs.append(jnp.maximum(xs[j], v))
            return tuple(new_ms), tuple(new_xs)

        mns, mxs = _mm

    minv, maxv = mns[0], mxs[0]
    for j in range(1, _U):
        minv = jnp.minimum(minv, mns[j])
        maxv = jnp.maximum(maxv, mxs[j])
    stage[...] = minv
    pltpu.sync_copy(stage, pmin_hbm.at[wid])
    stage[...] = maxv
    pltpu.sync_copy(stage, pmax_hbm.at[wid])


def _hist_body(x_hbm, pmin_hbm, pmax_hbm, xout_hbm, phist_hbm,
               buf0, buf1, mmbuf, hist, out_slot, lsem0, lsem1, ssem0, ssem1):
    wid = lax.axis_index("c") * _NS + lax.axis_index("s")
    nrows = x_hbm.shape[0]
    rows_w = nrows // _NW
    rows_c = _CHUNK // 2048
    base = wid * rows_w
    n_chunks = rows_w // rows_c
    lsem = (lsem0, lsem1)
    ssem = (ssem0, ssem1)
    bufs = (buf0, buf1)

    def load(ci, s):
        sl = pl.ds(base + ci * rows_c, rows_c)
        return pltpu.make_async_copy(x_hbm.at[sl, :], bufs[s], lsem[s])

    def store(ci, s):
        sl = pl.ds(base + ci * rows_c, rows_c)
        return pltpu.make_async_copy(bufs[s], xout_hbm.at[sl, :], ssem[s])

    load(0, 0).start()

    pltpu.sync_copy(pmin_hbm, mmbuf.at[0])
    pltpu.sync_copy(pmax_hbm, mmbuf.at[1])

    def mmstep(i, c):
        mn, mx = c
        return (jnp.minimum(mn, mmbuf[0, pl.ds(i * _L, _L)]),
                jnp.maximum(mx, mmbuf[1, pl.ds(i * _L, _L)]))

    minv, maxv = lax.fori_loop(
        0, _NW, mmstep,
        (jnp.full((_L,), jnp.inf, jnp.float32),
         jnp.full((_L,), -jnp.inf, jnp.float32)))
    xmin, xmax = minv[0], maxv[0]
    for j in range(1, _L):
        xmin = jnp.minimum(xmin, minv[j])
        xmax = jnp.maximum(xmax, maxv[j])
    rng = jnp.where(xmax > xmin, xmax - xmin, jnp.float32(1.0))
    rngv = jnp.full((_L,), 1.0, jnp.float32) * rng
    scale = jnp.full((_L,), float(_BINS), jnp.float32) / rngv
    noff = -(xmin * scale)

    def zstep(i, _):
        hist[pl.ds(i * _L, _L)] = jnp.zeros((_L,), jnp.float32)
        return 0

    lax.fori_loop(0, _L * _STRIDE // _L, zstep, 0)

    ones = jnp.ones((_L,), jnp.float32)
    lane_base = lax.broadcasted_iota(jnp.int32, (_L,), 0) * _STRIDE + 1

    for ci in range(n_chunks):
        s = ci & 1
        load(ci, s).wait()
        store(ci, s).start()
        if ci + 1 < n_chunks:
            if ci >= 1:
                store(ci - 1, 1 - s).wait()
            load(ci + 1, 1 - s).start()

        @plsc.parallel_loop(0, _CHUNK // _L, 1, unroll=_U)
        def _sc(i):
            v = bufs[s][i >> 7, pl.ds((i & 127) * _L, _L)]
            t = v * scale + noff
            idx = t.astype(jnp.int32) + lane_base
            plsc.addupdate_scatter(hist, [idx], ones)

    rowv = lax.broadcasted_iota(jnp.int32, (_L,), 0) * _STRIDE
    gv = plsc.load_gather(hist, [rowv])
    plsc.addupdate_scatter(hist, [rowv + 1], gv)
    ov = plsc.load_gather(hist, [rowv + _BINS + 1])
    plsc.addupdate_scatter(hist, [rowv + _BINS], ov)

    def rstep(k, _):
        acc = hist[pl.ds(1 + k * _L, _L)]
        for l in range(1, _L):
            acc = acc + hist[pl.ds(l * _STRIDE + 1 + k * _L, _L)]
        out_slot[pl.ds(k * _L, _L)] = acc
        return 0

    store(n_chunks - 1, (n_chunks - 1) & 1).wait()
    store(n_chunks - 2, (n_chunks - 2) & 1).wait()
    lax.fori_loop(0, _BINS // _L, rstep, 0)
    pltpu.sync_copy(out_slot, phist_hbm.at[wid])


def _sc_minmax(x_flat):
    f = pl.kernel(
        _minmax_body,
        out_type=(jax.ShapeDtypeStruct((_NW, _L), jnp.float32),
                  jax.ShapeDtypeStruct((_NW, _L), jnp.float32)),
        mesh=_mesh(),
        scratch_types=[pltpu.VMEM((_CHUNK // 2048, 2048), jnp.float32),
                       pltpu.VMEM((_CHUNK // 2048, 2048), jnp.float32),
                       pltpu.VMEM((_L,), jnp.float32),
                       pltpu.SemaphoreType.DMA,
                       pltpu.SemaphoreType.DMA],
        compiler_params=pltpu.CompilerParams(needs_layout_passes=False, use_tc_tiling_on_sc=True),
    )
    return f(x_flat)


def _sc_hist(x_flat, pmin, pmax):
    f = pl.kernel(
        _hist_body,
        out_type=(jax.ShapeDtypeStruct(x_flat.shape, jnp.float32),
                  jax.ShapeDtypeStruct((_NW, _BINS), jnp.float32)),
        mesh=_mesh(),
        scratch_types=[pltpu.VMEM((_CHUNK // 2048, 2048), jnp.float32),
                       pltpu.VMEM((_CHUNK // 2048, 2048), jnp.float32),
                       pltpu.VMEM((2, _NW * _L), jnp.float32),
                       pltpu.VMEM((_L * _STRIDE,), jnp.float32),
                       pltpu.VMEM((_BINS,), jnp.float32),
                       pltpu.SemaphoreType.DMA,
                       pltpu.SemaphoreType.DMA,
                       pltpu.SemaphoreType.DMA,
                       pltpu.SemaphoreType.DMA],
        compiler_params=pltpu.CompilerParams(needs_layout_passes=False, use_tc_tiling_on_sc=True),
    )
    return f(x_flat, pmin.reshape(-1), pmax.reshape(-1))


def kernel(x):
    x_flat = x.reshape(-1, x.shape[-1])
    pmin, pmax = _sc_minmax(x_flat)
    x_out, phist = _sc_hist(x_flat, pmin, pmax)
    xmin = jnp.min(pmin)
    xmax = jnp.max(pmax)
    hist = jnp.sum(phist, axis=0)
    return (x_out.reshape(x.shape), xmin, xmax, hist)


# speedup vs baseline: 1.0987x; 1.0919x over previous
import jax
import jax.numpy as jnp
from jax import lax
from jax.experimental import pallas as pl
from jax.experimental.pallas import tpu as pltpu
from jax.experimental.pallas import tpu_sc as plsc

_BINS = 2048
_NC = 2
_NS = 16
_NW = _NC * _NS
_L = 16
_CHUNK = 32768
_U = 8
_STRIDE = _BINS + 2


def _mesh():
    return plsc.VectorSubcoreMesh(core_axis_name="c", subcore_axis_name="s",
                                  num_cores=_NC, num_subcores=_NS)


def _tc_minmax_body(x_ref, mn_ref, mx_ref, acc):
    i = pl.program_id(0)
    blk = x_ref[...]
    bmn = jnp.min(blk)
    bmx = jnp.max(blk)

    @pl.when(i == 0)
    def _():
        acc[0] = bmn
        acc[1] = bmx

    @pl.when(i > 0)
    def _():
        acc[0] = jnp.minimum(acc[0], bmn)
        acc[1] = jnp.maximum(acc[1], bmx)

    @pl.when(i == pl.num_programs(0) - 1)
    def _():
        mn_ref[...] = jnp.full((8, 128), acc[0], jnp.float32)
        mx_ref[...] = jnp.full((8, 128), acc[1], jnp.float32)


def _tc_minmax(x2d):
    nrows = x2d.shape[0]
    br = 512
    f = pl.pallas_call(
        _tc_minmax_body,
        grid=(nrows // br,),
        in_specs=[pl.BlockSpec((br, 2048), lambda i: (i, 0))],
        out_specs=[pl.BlockSpec((8, 128), lambda i: (0, 0)),
                   pl.BlockSpec((8, 128), lambda i: (0, 0))],
        out_shape=(jax.ShapeDtypeStruct((8, 128), jnp.float32),
                   jax.ShapeDtypeStruct((8, 128), jnp.float32)),
        scratch_shapes=[pltpu.SMEM((2,), jnp.float32)],
        compiler_params=pltpu.CompilerParams(
            dimension_semantics=("arbitrary",)),
    )
    return f(x2d)


def _hist_body(x_hbm, pmn_hbm, pmx_hbm, xout_hbm, phist_hbm,
               buf0, buf1, mn_v, mx_v, hist, out_slot,
               lsem0, lsem1, ssem0, ssem1):
    wid = lax.axis_index("c") * _NS + lax.axis_index("s")
    nrows = x_hbm.shape[0]
    rows_w = nrows // _NW
    rows_c = _CHUNK // 2048
    base = wid * rows_w
    n_chunks = rows_w // rows_c
    lsem = (lsem0, lsem1)
    ssem = (ssem0, ssem1)
    bufs = (buf0, buf1)

    def load(ci, s):
        sl = pl.ds(base + ci * rows_c, rows_c)
        return pltpu.make_async_copy(x_hbm.at[sl, :], bufs[s], lsem[s])

    def store(ci, s):
        sl = pl.ds(base + ci * rows_c, rows_c)
        return pltpu.make_async_copy(bufs[s], xout_hbm.at[sl, :], ssem[s])

    load(0, 0).start()

    pltpu.sync_copy(pmn_hbm, mn_v)
    pltpu.sync_copy(pmx_hbm, mx_v)
    xmin = mn_v[0, pl.ds(0, _L)][0]
    xmax = mx_v[0, pl.ds(0, _L)][0]
    rng = jnp.where(xmax > xmin, xmax - xmin, jnp.float32(1.0))
    rngv = jnp.full((_L,), 1.0, jnp.float32) * rng
    scale = jnp.full((_L,), float(_BINS), jnp.float32) / rngv
    noff = -(xmin * scale)

    def zstep(i, _):
        hist[pl.ds(i * _L, _L)] = jnp.zeros((_L,), jnp.float32)
        return 0

    lax.fori_loop(0, _L * _STRIDE // _L, zstep, 0)

    ones = jnp.ones((_L,), jnp.float32)
    lane_base = lax.broadcasted_iota(jnp.int32, (_L,), 0) * _STRIDE + 1

    for ci in range(n_chunks):
        s = ci & 1
        load(ci, s).wait()
        store(ci, s).start()
        if ci + 1 < n_chunks:
            if ci >= 1:
                store(ci - 1, 1 - s).wait()
            load(ci + 1, 1 - s).start()

        @plsc.parallel_loop(0, _CHUNK // _L, 1, unroll=_U)
        def _sc(i):
            v = bufs[s][i >> 7, pl.ds((i & 127) * _L, _L)]
            t = v * scale + noff
            idx = t.astype(jnp.int32) + lane_base
            plsc.addupdate_scatter(hist, [idx], ones)

    rowv = lax.broadcasted_iota(jnp.int32, (_L,), 0) * _STRIDE
    gv = plsc.load_gather(hist, [rowv])
    plsc.addupdate_scatter(hist, [rowv + 1], gv)
    ov = plsc.load_gather(hist, [rowv + _BINS + 1])
    plsc.addupdate_scatter(hist, [rowv + _BINS], ov)

    def rstep(k, _):
        acc = hist[pl.ds(1 + k * _L, _L)]
        for l in range(1, _L):
            acc = acc + hist[pl.ds(l * _STRIDE + 1 + k * _L, _L)]
        out_slot[pl.ds(k * _L, _L)] = acc
        return 0

    store(n_chunks - 1, (n_chunks - 1) & 1).wait()
    store(n_chunks - 2, (n_chunks - 2) & 1).wait()
    lax.fori_loop(0, _BINS // _L, rstep, 0)
    pltpu.sync_copy(out_slot, phist_hbm.at[wid])


def _sc_hist(x_flat, pmn, pmx):
    f = pl.kernel(
        _hist_body,
        out_type=(jax.ShapeDtypeStruct(x_flat.shape, jnp.float32),
                  jax.ShapeDtypeStruct((_NW, _BINS), jnp.float32)),
        mesh=_mesh(),
        scratch_types=[pltpu.VMEM((_CHUNK // 2048, 2048), jnp.float32),
                       pltpu.VMEM((_CHUNK // 2048, 2048), jnp.float32),
                       pltpu.VMEM((8, 128), jnp.float32),
                       pltpu.VMEM((8, 128), jnp.float32),
                       pltpu.VMEM((_L * _STRIDE,), jnp.float32),
                       pltpu.VMEM((_BINS,), jnp.float32),
                       pltpu.SemaphoreType.DMA,
                       pltpu.SemaphoreType.DMA,
                       pltpu.SemaphoreType.DMA,
                       pltpu.SemaphoreType.DMA],
        compiler_params=pltpu.CompilerParams(needs_layout_passes=False, use_tc_tiling_on_sc=True),
    )
    return f(x_flat, pmn, pmx)


def kernel(x):
    x_flat = x.reshape(-1, x.shape[-1])
    pmn, pmx = _tc_minmax(x_flat)
    x_out, phist = _sc_hist(x_flat, pmn, pmx)
    hist = jnp.sum(phist, axis=0)
    return (x_out.reshape(x.shape), pmn[0, 0], pmx[0, 0], hist)
